# baseline (speedup 1.0000x reference)
import functools

import jax
import jax.numpy as jnp
from jax import lax
from jax.experimental import pallas as pl
from jax.experimental.pallas import tpu as pltpu
from jax.experimental.pallas import tpu_sc as plsc

N = 10000
E = 320000
D = 128
DH = D // 2

NC = 2
NS = 16
NPAD = 10240
RPT = NPAD // NS
EPT = E // NS
KI = 128
NCH = -(-EPT // KI)
EPP = NCH * KI
NPADE = EPP - EPT
WB = 80
_MESH = plsc.VectorSubcoreMesh(
    core_axis_name="c", subcore_axis_name="s", num_cores=NC, num_subcores=NS
)
_F32 = jnp.float32


@functools.partial(
    pl.kernel,
    out_type=jax.ShapeDtypeStruct((NC, NPAD), _F32),
    mesh=_MESH,
    scratch_types=[
        pltpu.VMEM((NCH, KI), jnp.int32),
        pltpu.VMEM((KI,), _F32),
        pltpu.VMEM((RPT,), _F32),
        pltpu.VMEM_SHARED((NPAD,), _F32),
        pltpu.SemaphoreType.DMA,
        pltpu.SemaphoreType.DMA,
        pltpu.SemaphoreType.DMA,
        pltpu.SemaphoreType.DMA,
    ],
)
def _deg_kernel(stbl_hbm, out_hbm, idx_v, ones_v, zb_v, deg_sh,
                sem0, sem1, sem2, sem3):
    cid = lax.axis_index("c")
    sid = lax.axis_index("s")
    for i in range(KI // 16):
        ones_v[pl.ds(i * 16, 16)] = jnp.ones((16,), _F32)
    for i in range(RPT // 16):
        zb_v[pl.ds(i * 16, 16)] = jnp.zeros((16,), _F32)
    pltpu.sync_copy(zb_v, deg_sh.at[pl.ds(sid * RPT, RPT)])
    plsc.subcore_barrier()

    @pl.when(cid == 0)
    def _():
        pltpu.sync_copy(stbl_hbm.at[0, sid], idx_v)

    @pl.when(cid == 1)
    def _():
        pltpu.sync_copy(stbl_hbm.at[1, sid], idx_v)

    sems = (sem0, sem1, sem2, sem3)

    @pl.loop(0, NCH - (NCH % 4), step=4)
    def _(j):
        handles = [
            pltpu.async_copy(
                ones_v, deg_sh.at[idx_v.at[j + s]], sems[s], add=True)
            for s in range(4)
        ]
        for h in handles:
            h.wait()

    for s in range(NCH % 4):
        j = NCH - (NCH % 4) + s
        pltpu.sync_copy(ones_v, deg_sh.at[idx_v.at[j]], add=True)

    plsc.subcore_barrier()
    pltpu.sync_copy(deg_sh.at[pl.ds(sid * RPT, RPT)], zb_v)

    @pl.when(cid == 0)
    def _():
        pltpu.sync_copy(zb_v, out_hbm.at[0, pl.ds(sid * RPT, RPT)])

    @pl.when(cid == 1)
    def _():
        pltpu.sync_copy(zb_v, out_hbm.at[1, pl.ds(sid * RPT, RPT)])


@functools.partial(
    pl.kernel,
    out_type=tuple(
        jax.ShapeDtypeStruct((NPAD, DH), _F32) for _ in range(4)
    ),
    mesh=_MESH,
    scratch_types=[
        pltpu.VMEM((NCH, KI), jnp.int32),
        pltpu.VMEM((NCH, KI), jnp.int32),
        pltpu.VMEM((KI, DH), _F32),
        pltpu.VMEM((KI, DH), _F32),
        pltpu.VMEM((KI, DH), _F32),
        pltpu.VMEM((KI, DH), _F32),
        pltpu.VMEM((WB, DH), _F32),
        pltpu.VMEM_SHARED((NPAD, DH), _F32),
        pltpu.SemaphoreType.DMA,
        pltpu.SemaphoreType.DMA,
        pltpu.SemaphoreType.DMA,
        pltpu.SemaphoreType.DMA,
    ],
    compiler_params=pltpu.CompilerParams(use_tc_tiling_on_sc=False),
)
def _edge_kernel(yf0_hbm, yf1_hbm, yb0_hbm, yb1_hbm, gtbl_hbm, stbl_hbm,
                 af0_hbm, af1_hbm, ab0_hbm, ab1_hbm,
                 gidx_v, sidx_v, rows0_v, rows1_v, rows2_v, rows3_v,
                 zb_v, acc_sh, sem0, sem1, sem2, sem3):
    cid = lax.axis_index("c")
    sid = lax.axis_index("s")
    for i in range(WB):
        for j in range(DH // 16):
            zb_v[i, pl.ds(j * 16, 16)] = jnp.zeros((16,), _F32)

    def one_task(trow, tables, outs):
        for t in range(RPT // WB):
            pltpu.sync_copy(zb_v, acc_sh.at[pl.ds(sid * RPT + t * WB, WB)])
        plsc.subcore_barrier()
        pltpu.sync_copy(gtbl_hbm.at[trow, sid], gidx_v)
        pltpu.sync_copy(stbl_hbm.at[trow, sid], sidx_v)

        bufs = (rows0_v, rows1_v, rows2_v, rows3_v)
        sems = (sem0, sem1, sem2, sem3)

        def run(table_hbm):
            for s in range(4):
                pltpu.async_copy(table_hbm.at[gidx_v.at[s]], bufs[s], sems[s])

            @pl.loop(0, NCH - (NCH % 4), step=4)
            def _(j):
                for s in range(4):
                    pltpu.make_async_copy(
                        table_hbm.at[gidx_v.at[j + s]], bufs[s], sems[s]).wait()
                    pltpu.sync_copy(
                        bufs[s], acc_sh.at[sidx_v.at[j + s]], add=True)

                    @pl.when(j + s + 4 < NCH)
                    def _():
                        pltpu.async_copy(
                            table_hbm.at[gidx_v.at[j + s + 4]], bufs[s], sems[s])

            for s in range(NCH % 4):
                j = NCH - (NCH % 4) + s
                pltpu.make_async_copy(
                    table_hbm.at[gidx_v.at[j]], bufs[s], sems[s]).wait()
                pltpu.sync_copy(bufs[s], acc_sh.at[sidx_v.at[j]], add=True)

        @pl.when(cid == 0)
        def _():
            run(tables[0])

        @pl.when(cid == 1)
        def _():
            run(tables[1])

        plsc.subcore_barrier()

        def writeback(out_hbm):
            for t in range(RPT // WB):
                off = sid * RPT + t * WB
                pltpu.sync_copy(acc_sh.at[pl.ds(off, WB)],
                                rows0_v.at[pl.ds(0, WB)])
                pltpu.sync_copy(rows0_v.at[pl.ds(0, WB)],
                                out_hbm.at[pl.ds(off, WB)])

        @pl.when(cid == 0)
        def _():
            writeback(outs[0])

        @pl.when(cid == 1)
        def _():
            writeback(outs[1])

    one_task(0, (yf0_hbm, yf1_hbm), (af0_hbm, af1_hbm))
    one_task(1, (yb0_hbm, yb1_hbm), (ab0_hbm, ab1_hbm))


def _mm_body(x_ref, wf_ref, wb_ref, deg_ref,
             yf0_ref, yf1_ref, yb0_ref, yb1_ref):
    dinv = lax.rsqrt(deg_ref[...] + 1.0)
    x = x_ref[...]
    yf = jnp.dot(x, wf_ref[...], preferred_element_type=_F32) * dinv[:, 0:1]
    yb = jnp.dot(x, wb_ref[...], preferred_element_type=_F32) * dinv[:, 1:2]
    yf0_ref[...] = yf[:, :DH]
    yf1_ref[...] = yf[:, DH:]
    yb0_ref[...] = yb[:, :DH]
    yb1_ref[...] = yb[:, DH:]


BN = 1000
_G = N // BN

_mm_call = pl.pallas_call(
    _mm_body,
    grid=(_G,),
    in_specs=[
        pl.BlockSpec((BN, D), lambda i: (i, 0)),
        pl.BlockSpec((D, D), lambda i: (0, 0)),
        pl.BlockSpec((D, D), lambda i: (0, 0)),
        pl.BlockSpec((BN, 2), lambda i: (i, 0)),
    ],
    out_specs=tuple(pl.BlockSpec((BN, DH), lambda i: (i, 0)) for _ in range(4)),
    out_shape=tuple(jax.ShapeDtypeStruct((NPAD, DH), _F32) for _ in range(4)),
)


def _fin_body(af0_ref, af1_ref, ab0_ref, ab1_ref,
              yf0_ref, yf1_ref, yb0_ref, yb1_ref, deg_ref, b_ref, out_ref):
    dinv = lax.rsqrt(deg_ref[...] + 1.0)
    accf = jnp.concatenate([af0_ref[...], af1_ref[...]], axis=1)
    accb = jnp.concatenate([ab0_ref[...], ab1_ref[...]], axis=1)
    yf = jnp.concatenate([yf0_ref[...], yf1_ref[...]], axis=1)
    yb = jnp.concatenate([yb0_ref[...], yb1_ref[...]], axis=1)
    out = (dinv[:, 0:1] * (accf + yf) + dinv[:, 1:2] * (accb + yb) + b_ref[...])
    out_ref[...] = jnp.maximum(out, 0.0)


_fin_call = pl.pallas_call(
    _fin_body,
    grid=(_G,),
    in_specs=[pl.BlockSpec((BN, DH), lambda i: (i, 0)) for _ in range(8)]
    + [
        pl.BlockSpec((BN, 2), lambda i: (i, 0)),
        pl.BlockSpec((1, D), lambda i: (0, 0)),
    ],
    out_specs=pl.BlockSpec((BN, D), lambda i: (i, 0)),
    out_shape=jax.ShapeDtypeStruct((N, D), _F32),
)


@jax.jit
def kernel(x, edge_index, Wf, bf, Wb, bb):
    src = edge_index[0].reshape(NS, EPT)
    dst = edge_index[1].reshape(NS, EPT)
    gpad = jnp.zeros((NS, NPADE), jnp.int32)
    spad = (N + (jnp.arange(NS * NPADE, dtype=jnp.int32) % (NPAD - N))
            ).reshape(NS, NPADE)
    gtbl = jnp.stack([
        jnp.concatenate([src, gpad], axis=1),
        jnp.concatenate([dst, gpad], axis=1),
    ]).reshape(2, NS, NCH, KI)
    stbl = jnp.stack([
        jnp.concatenate([dst, spad], axis=1),
        jnp.concatenate([src, spad], axis=1),
    ]).reshape(2, NS, NCH, KI)
    deg2 = _deg_kernel(stbl)
    deg_t = deg2.T
    yf0, yf1, yb0, yb1 = _mm_call(x, Wf, Wb, deg_t)
    af0, af1, ab0, ab1 = _edge_kernel(yf0, yf1, yb0, yb1, gtbl, stbl)
    bias = (bf + bb)[None, :]
    return _fin_call(af0, af1, ab0, ab1, yf0, yf1, yb0, yb1, deg_t, bias)


# speedup vs baseline: 1.2257x; 1.2257x over previous
import functools

import jax
import jax.numpy as jnp
from jax import lax
from jax.experimental import pallas as pl
from jax.experimental.pallas import tpu as pltpu
from jax.experimental.pallas import tpu_sc as plsc

N = 10000
E = 320000
D = 128
DH = D // 2

NC = 2
NS = 16
NPAD = 10240
RPT = NPAD // NS
EPT = E // NS
KI = 80
NCH = EPT // KI
WB = 80
_MESH = plsc.VectorSubcoreMesh(
    core_axis_name="c", subcore_axis_name="s", num_cores=NC, num_subcores=NS
)
_F32 = jnp.float32


@functools.partial(
    pl.kernel,
    out_type=jax.ShapeDtypeStruct((NC, NPAD), _F32),
    mesh=_MESH,
    scratch_types=[
        pltpu.VMEM((NCH, KI), jnp.int32),
        pltpu.VMEM((KI,), _F32),
        pltpu.VMEM((RPT,), _F32),
        pltpu.VMEM_SHARED((NPAD,), _F32),
        pltpu.SemaphoreType.DMA,
        pltpu.SemaphoreType.DMA,
        pltpu.SemaphoreType.DMA,
        pltpu.SemaphoreType.DMA,
    ],
)
def _deg_kernel(ei_hbm, out_hbm, idx_v, ones_v, zb_v, deg_sh,
                sem0, sem1, sem2, sem3):
    cid = lax.axis_index("c")
    sid = lax.axis_index("s")
    for i in range(KI // 16):
        ones_v[pl.ds(i * 16, 16)] = jnp.ones((16,), _F32)
    for i in range(RPT // 16):
        zb_v[pl.ds(i * 16, 16)] = jnp.zeros((16,), _F32)
    pltpu.sync_copy(zb_v, deg_sh.at[pl.ds(sid * RPT, RPT)])
    plsc.subcore_barrier()

    @pl.when(cid == 0)
    def _():
        pltpu.sync_copy(ei_hbm.at[1, sid], idx_v)

    @pl.when(cid == 1)
    def _():
        pltpu.sync_copy(ei_hbm.at[0, sid], idx_v)

    sems = (sem0, sem1, sem2, sem3)

    @pl.loop(0, NCH - (NCH % 4), step=4)
    def _(j):
        handles = [
            pltpu.async_copy(
                ones_v, deg_sh.at[idx_v.at[j + s]], sems[s], add=True)
            for s in range(4)
        ]
        for h in handles:
            h.wait()

    for s in range(NCH % 4):
        j = NCH - (NCH % 4) + s
        pltpu.sync_copy(ones_v, deg_sh.at[idx_v.at[j]], add=True)

    plsc.subcore_barrier()
    pltpu.sync_copy(deg_sh.at[pl.ds(sid * RPT, RPT)], zb_v)

    @pl.when(cid == 0)
    def _():
        pltpu.sync_copy(zb_v, out_hbm.at[0, pl.ds(sid * RPT, RPT)])

    @pl.when(cid == 1)
    def _():
        pltpu.sync_copy(zb_v, out_hbm.at[1, pl.ds(sid * RPT, RPT)])


@functools.partial(
    pl.kernel,
    out_type=tuple(
        jax.ShapeDtypeStruct((NPAD, DH), _F32) for _ in range(4)
    ),
    mesh=_MESH,
    scratch_types=[
        pltpu.VMEM((NCH, KI), jnp.int32),
        pltpu.VMEM((NCH, KI), jnp.int32),
        pltpu.VMEM((KI, DH), _F32),
        pltpu.VMEM((KI, DH), _F32),
        pltpu.VMEM((KI, DH), _F32),
        pltpu.VMEM((KI, DH), _F32),
        pltpu.VMEM((WB, DH), _F32),
        pltpu.VMEM_SHARED((NPAD, DH), _F32),
        pltpu.SemaphoreType.DMA,
        pltpu.SemaphoreType.DMA,
        pltpu.SemaphoreType.DMA,
        pltpu.SemaphoreType.DMA,
    ],
    compiler_params=pltpu.CompilerParams(use_tc_tiling_on_sc=False),
)
def _edge_kernel(yf0_hbm, yf1_hbm, yb0_hbm, yb1_hbm, ei_hbm,
                 af0_hbm, af1_hbm, ab0_hbm, ab1_hbm,
                 gidx_v, sidx_v, rows0_v, rows1_v, rows2_v, rows3_v,
                 zb_v, acc_sh, sem0, sem1, sem2, sem3):
    cid = lax.axis_index("c")
    sid = lax.axis_index("s")
    for i in range(WB):
        for j in range(DH // 16):
            zb_v[i, pl.ds(j * 16, 16)] = jnp.zeros((16,), _F32)

    def one_task(grow, srow, tables, outs):
        for t in range(RPT // WB):
            pltpu.sync_copy(zb_v, acc_sh.at[pl.ds(sid * RPT + t * WB, WB)])
        plsc.subcore_barrier()
        pltpu.sync_copy(ei_hbm.at[grow, sid], gidx_v)
        pltpu.sync_copy(ei_hbm.at[srow, sid], sidx_v)

        bufs = (rows0_v, rows1_v, rows2_v, rows3_v)
        sems = (sem0, sem1, sem2, sem3)

        def run(table_hbm):
            for s in range(4):
                pltpu.async_copy(table_hbm.at[gidx_v.at[s]], bufs[s], sems[s])

            @pl.loop(0, NCH - (NCH % 4), step=4)
            def _(j):
                for s in range(4):
                    pltpu.make_async_copy(
                        table_hbm.at[gidx_v.at[j + s]], bufs[s], sems[s]).wait()
                    pltpu.sync_copy(
                        bufs[s], acc_sh.at[sidx_v.at[j + s]], add=True)

                    @pl.when(j + s + 4 < NCH)
                    def _():
                        pltpu.async_copy(
                            table_hbm.at[gidx_v.at[j + s + 4]], bufs[s], sems[s])

            for s in range(NCH % 4):
                j = NCH - (NCH % 4) + s
                pltpu.make_async_copy(
                    table_hbm.at[gidx_v.at[j]], bufs[s], sems[s]).wait()
                pltpu.sync_copy(bufs[s], acc_sh.at[sidx_v.at[j]], add=True)

        @pl.when(cid == 0)
        def _():
            run(tables[0])

        @pl.when(cid == 1)
        def _():
            run(tables[1])

        plsc.subcore_barrier()

        def writeback(out_hbm):
            for t in range(RPT // WB):
                off = sid * RPT + t * WB
                pltpu.sync_copy(acc_sh.at[pl.ds(off, WB)], rows0_v)
                pltpu.sync_copy(rows0_v, out_hbm.at[pl.ds(off, WB)])

        @pl.when(cid == 0)
        def _():
            writeback(outs[0])

        @pl.when(cid == 1)
        def _():
            writeback(outs[1])

    one_task(0, 1, (yf0_hbm, yf1_hbm), (af0_hbm, af1_hbm))
    one_task(1, 0, (yb0_hbm, yb1_hbm), (ab0_hbm, ab1_hbm))


def _mm_body(x_ref, wf_ref, wb_ref, xwf_ref, xwb_ref):
    x = x_ref[...]
    xwf_ref[...] = jnp.dot(x, wf_ref[...], preferred_element_type=_F32)
    xwb_ref[...] = jnp.dot(x, wb_ref[...], preferred_element_type=_F32)


def _scale_body(xwf_ref, xwb_ref, deg_ref,
                yf0_ref, yf1_ref, yb0_ref, yb1_ref):
    dinv = lax.rsqrt(deg_ref[...] + 1.0)
    yf = xwf_ref[...] * dinv[:, 0:1]
    yb = xwb_ref[...] * dinv[:, 1:2]
    yf0_ref[...] = yf[:, :DH]
    yf1_ref[...] = yf[:, DH:]
    yb0_ref[...] = yb[:, :DH]
    yb1_ref[...] = yb[:, DH:]


BN = 1000
_G = N // BN

_mm_call = pl.pallas_call(
    _mm_body,
    grid=(_G,),
    in_specs=[
        pl.BlockSpec((BN, D), lambda i: (i, 0)),
        pl.BlockSpec((D, D), lambda i: (0, 0)),
        pl.BlockSpec((D, D), lambda i: (0, 0)),
    ],
    out_specs=tuple(pl.BlockSpec((BN, D), lambda i: (i, 0)) for _ in range(2)),
    out_shape=tuple(jax.ShapeDtypeStruct((N, D), _F32) for _ in range(2)),
)

_scale_call = pl.pallas_call(
    _scale_body,
    grid=(_G,),
    in_specs=[
        pl.BlockSpec((BN, D), lambda i: (i, 0)),
        pl.BlockSpec((BN, D), lambda i: (i, 0)),
        pl.BlockSpec((BN, 2), lambda i: (i, 0)),
    ],
    out_specs=tuple(pl.BlockSpec((BN, DH), lambda i: (i, 0)) for _ in range(4)),
    out_shape=tuple(jax.ShapeDtypeStruct((NPAD, DH), _F32) for _ in range(4)),
)


def _fin_body(af0_ref, af1_ref, ab0_ref, ab1_ref,
              yf0_ref, yf1_ref, yb0_ref, yb1_ref, deg_ref, b_ref, out_ref):
    dinv = lax.rsqrt(deg_ref[...] + 1.0)
    accf = jnp.concatenate([af0_ref[...], af1_ref[...]], axis=1)
    accb = jnp.concatenate([ab0_ref[...], ab1_ref[...]], axis=1)
    yf = jnp.concatenate([yf0_ref[...], yf1_ref[...]], axis=1)
    yb = jnp.concatenate([yb0_ref[...], yb1_ref[...]], axis=1)
    out = (dinv[:, 0:1] * (accf + yf) + dinv[:, 1:2] * (accb + yb) + b_ref[...])
    out_ref[...] = jnp.maximum(out, 0.0)


_fin_call = pl.pallas_call(
    _fin_body,
    grid=(_G,),
    in_specs=[pl.BlockSpec((BN, DH), lambda i: (i, 0)) for _ in range(8)]
    + [
        pl.BlockSpec((BN, 2), lambda i: (i, 0)),
        pl.BlockSpec((1, D), lambda i: (0, 0)),
    ],
    out_specs=pl.BlockSpec((BN, D), lambda i: (i, 0)),
    out_shape=jax.ShapeDtypeStruct((N, D), _F32),
)


@jax.jit
def kernel(x, edge_index, Wf, bf, Wb, bb):
    ei4 = edge_index.reshape(NC, NS, NCH, KI)
    deg2 = _deg_kernel(ei4)
    deg_t = deg2.T
    xwf, xwb = _mm_call(x, Wf, Wb)
    yf0, yf1, yb0, yb1 = _scale_call(xwf, xwb, deg_t)
    af0, af1, ab0, ab1 = _edge_kernel(yf0, yf1, yb0, yb1, ei4)
    bias = (bf + bb)[None, :]
    return _fin_call(af0, af1, ab0, ab1, yf0, yf1, yb0, yb1, deg_t, bias)


# speedup vs baseline: 1.2342x; 1.0069x over previous
import functools

import jax
import jax.numpy as jnp
from jax import lax
from jax.experimental import pallas as pl
from jax.experimental.pallas import tpu as pltpu
from jax.experimental.pallas import tpu_sc as plsc

N = 10000
E = 320000
D = 128
DH = D // 2

NC = 2
NS = 16
NPAD = 10240
RPT = NPAD // NS
EPT = E // NS
KI = 80
NCH = EPT // KI
WB = 80
_MESH = plsc.VectorSubcoreMesh(
    core_axis_name="c", subcore_axis_name="s", num_cores=NC, num_subcores=NS
)
_F32 = jnp.float32


@functools.partial(
    pl.kernel,
    out_type=jax.ShapeDtypeStruct((NC, NPAD), _F32),
    mesh=_MESH,
    scratch_types=[
        pltpu.VMEM((NCH, KI), jnp.int32),
        pltpu.VMEM((KI,), _F32),
        pltpu.VMEM((RPT,), _F32),
        pltpu.VMEM_SHARED((NPAD,), _F32),
        pltpu.SemaphoreType.DMA,
        pltpu.SemaphoreType.DMA,
        pltpu.SemaphoreType.DMA,
        pltpu.SemaphoreType.DMA,
    ],
)
def _deg_kernel(ei_hbm, out_hbm, idx_v, ones_v, zb_v, deg_sh,
                sem0, sem1, sem2, sem3):
    cid = lax.axis_index("c")
    sid = lax.axis_index("s")
    for i in range(KI // 16):
        ones_v[pl.ds(i * 16, 16)] = jnp.ones((16,), _F32)
    for i in range(RPT // 16):
        zb_v[pl.ds(i * 16, 16)] = jnp.zeros((16,), _F32)
    pltpu.sync_copy(zb_v, deg_sh.at[pl.ds(sid * RPT, RPT)])
    plsc.subcore_barrier()

    @pl.when(cid == 0)
    def _():
        pltpu.sync_copy(ei_hbm.at[1, sid], idx_v)

    @pl.when(cid == 1)
    def _():
        pltpu.sync_copy(ei_hbm.at[0, sid], idx_v)

    sems = (sem0, sem1, sem2, sem3)

    @pl.loop(0, NCH - (NCH % 4), step=4)
    def _(j):
        handles = [
            pltpu.async_copy(
                ones_v, deg_sh.at[idx_v.at[j + s]], sems[s], add=True)
            for s in range(4)
        ]
        for h in handles:
            h.wait()

    for s in range(NCH % 4):
        j = NCH - (NCH % 4) + s
        pltpu.sync_copy(ones_v, deg_sh.at[idx_v.at[j]], add=True)

    plsc.subcore_barrier()
    pltpu.sync_copy(deg_sh.at[pl.ds(sid * RPT, RPT)], zb_v)

    @pl.when(cid == 0)
    def _():
        pltpu.sync_copy(zb_v, out_hbm.at[0, pl.ds(sid * RPT, RPT)])

    @pl.when(cid == 1)
    def _():
        pltpu.sync_copy(zb_v, out_hbm.at[1, pl.ds(sid * RPT, RPT)])


@functools.partial(
    pl.kernel,
    out_type=tuple(
        jax.ShapeDtypeStruct((NPAD, DH), _F32) for _ in range(4)
    ),
    mesh=_MESH,
    scratch_types=[
        pltpu.VMEM((NCH, KI), jnp.int32),
        pltpu.VMEM((NCH, KI), jnp.int32),
        pltpu.VMEM((KI, DH), _F32),
        pltpu.VMEM((KI, DH), _F32),
        pltpu.VMEM((KI, DH), _F32),
        pltpu.VMEM((KI, DH), _F32),
        pltpu.VMEM((WB, DH), _F32),
        pltpu.VMEM_SHARED((NPAD, DH), _F32),
        pltpu.SemaphoreType.DMA,
        pltpu.SemaphoreType.DMA,
        pltpu.SemaphoreType.DMA,
        pltpu.SemaphoreType.DMA,
    ],
    compiler_params=pltpu.CompilerParams(use_tc_tiling_on_sc=False),
)
def _edge_kernel(yf0_hbm, yf1_hbm, yb0_hbm, yb1_hbm, ei_hbm,
                 af0_hbm, af1_hbm, ab0_hbm, ab1_hbm,
                 gidx_v, sidx_v, rows0_v, rows1_v, rows2_v, rows3_v,
                 zb_v, acc_sh, sem0, sem1, sem2, sem3):
    cid = lax.axis_index("c")
    sid = lax.axis_index("s")
    for i in range(WB):
        for j in range(DH // 16):
            zb_v[i, pl.ds(j * 16, 16)] = jnp.zeros((16,), _F32)

    def one_task(grow, srow, tables, outs):
        for t in range(RPT // WB):
            pltpu.sync_copy(zb_v, acc_sh.at[pl.ds(sid * RPT + t * WB, WB)])
        plsc.subcore_barrier()
        pltpu.sync_copy(ei_hbm.at[grow, sid], gidx_v)
        pltpu.sync_copy(ei_hbm.at[srow, sid], sidx_v)

        bufs = (rows0_v, rows1_v, rows2_v, rows3_v)
        sems = (sem0, sem1, sem2, sem3)

        def run(table_hbm):
            for s in range(4):
                pltpu.async_copy(table_hbm.at[gidx_v.at[s]], bufs[s], sems[s])

            @pl.loop(0, NCH - (NCH % 4), step=4)
            def _(j):
                for s in range(4):
                    pltpu.make_async_copy(
                        table_hbm.at[gidx_v.at[j + s]], bufs[s], sems[s]).wait()
                    pltpu.sync_copy(
                        bufs[s], acc_sh.at[sidx_v.at[j + s]], add=True)

                    @pl.when(j + s + 4 < NCH)
                    def _():
                        pltpu.async_copy(
                            table_hbm.at[gidx_v.at[j + s + 4]], bufs[s], sems[s])

            for s in range(NCH % 4):
                j = NCH - (NCH % 4) + s
                pltpu.make_async_copy(
                    table_hbm.at[gidx_v.at[j]], bufs[s], sems[s]).wait()
                pltpu.sync_copy(bufs[s], acc_sh.at[sidx_v.at[j]], add=True)

        @pl.when(cid == 0)
        def _():
            run(tables[0])

        @pl.when(cid == 1)
        def _():
            run(tables[1])

        plsc.subcore_barrier()

        def writeback(out_hbm):
            for t in range(RPT // WB):
                off = sid * RPT + t * WB
                pltpu.sync_copy(acc_sh.at[pl.ds(off, WB)], rows0_v)
                pltpu.sync_copy(rows0_v, out_hbm.at[pl.ds(off, WB)])

        @pl.when(cid == 0)
        def _():
            writeback(outs[0])

        @pl.when(cid == 1)
        def _():
            writeback(outs[1])

    one_task(0, 1, (yf0_hbm, yf1_hbm), (af0_hbm, af1_hbm))
    one_task(1, 0, (yb0_hbm, yb1_hbm), (ab0_hbm, ab1_hbm))


def _mm_body(x_ref, wf_ref, wb_ref, deg_ref,
             yf0_ref, yf1_ref, yb0_ref, yb1_ref):
    dinv = lax.rsqrt(deg_ref[...] + 1.0)
    x = x_ref[...]
    yf = jnp.dot(x, wf_ref[...], preferred_element_type=_F32) * dinv[:, 0:1]
    yb = jnp.dot(x, wb_ref[...], preferred_element_type=_F32) * dinv[:, 1:2]
    yf0_ref[...] = yf[:, :DH]
    yf1_ref[...] = yf[:, DH:]
    yb0_ref[...] = yb[:, :DH]
    yb1_ref[...] = yb[:, DH:]


BN = 1000
_G = N // BN

_mm_call = pl.pallas_call(
    _mm_body,
    grid=(_G,),
    in_specs=[
        pl.BlockSpec((BN, D), lambda i: (i, 0)),
        pl.BlockSpec((D, D), lambda i: (0, 0)),
        pl.BlockSpec((D, D), lambda i: (0, 0)),
        pl.BlockSpec((BN, 2), lambda i: (i, 0)),
    ],
    out_specs=tuple(pl.BlockSpec((BN, DH), lambda i: (i, 0)) for _ in range(4)),
    out_shape=tuple(jax.ShapeDtypeStruct((NPAD, DH), _F32) for _ in range(4)),
)


def _fin_body(af0_ref, af1_ref, ab0_ref, ab1_ref,
              yf0_ref, yf1_ref, yb0_ref, yb1_ref, deg_ref, b_ref, out_ref):
    dinv = lax.rsqrt(deg_ref[...] + 1.0)
    accf = jnp.concatenate([af0_ref[...], af1_ref[...]], axis=1)
    accb = jnp.concatenate([ab0_ref[...], ab1_ref[...]], axis=1)
    yf = jnp.concatenate([yf0_ref[...], yf1_ref[...]], axis=1)
    yb = jnp.concatenate([yb0_ref[...], yb1_ref[...]], axis=1)
    out = (dinv[:, 0:1] * (accf + yf) + dinv[:, 1:2] * (accb + yb) + b_ref[...])
    out_ref[...] = jnp.maximum(out, 0.0)


_fin_call = pl.pallas_call(
    _fin_body,
    grid=(_G,),
    in_specs=[pl.BlockSpec((BN, DH), lambda i: (i, 0)) for _ in range(8)]
    + [
        pl.BlockSpec((BN, 2), lambda i: (i, 0)),
        pl.BlockSpec((1, D), lambda i: (0, 0)),
    ],
    out_specs=pl.BlockSpec((BN, D), lambda i: (i, 0)),
    out_shape=jax.ShapeDtypeStruct((N, D), _F32),
)


@jax.jit
def kernel(x, edge_index, Wf, bf, Wb, bb):
    ei4 = edge_index.reshape(NC, NS, NCH, KI)
    deg2 = _deg_kernel(ei4)
    deg_t = deg2.T
    yf0, yf1, yb0, yb1 = _mm_call(x, Wf, Wb, deg_t)
    af0, af1, ab0, ab1 = _edge_kernel(yf0, yf1, yb0, yb1, ei4)
    bias = (bf + bb)[None, :]
    return _fin_call(af0, af1, ab0, ab1, yf0, yf1, yb0, yb1, deg_t, bias)


# speedup vs baseline: 1.2666x; 1.0262x over previous
import functools

import jax
import jax.numpy as jnp
from jax import lax
from jax.experimental import pallas as pl
from jax.experimental.pallas import tpu as pltpu
from jax.experimental.pallas import tpu_sc as plsc

N = 10000
E = 320000
D = 128
DH = D // 2

NC = 2
NS = 16
NPAD = 10240
RPT = NPAD // NS
EPT = E // NS
KI = 80
NCH = EPT // KI
WB = 80
_MESH = plsc.VectorSubcoreMesh(
    core_axis_name="c", subcore_axis_name="s", num_cores=NC, num_subcores=NS
)
_F32 = jnp.float32


@functools.partial(
    pl.kernel,
    out_type=jax.ShapeDtypeStruct((NC, NPAD), _F32),
    mesh=_MESH,
    scratch_types=[
        pltpu.VMEM((NCH, KI), jnp.int32),
        pltpu.VMEM((KI,), _F32),
        pltpu.VMEM((RPT,), _F32),
        pltpu.VMEM_SHARED((NPAD,), _F32),
        pltpu.SemaphoreType.DMA,
        pltpu.SemaphoreType.DMA,
        pltpu.SemaphoreType.DMA,
        pltpu.SemaphoreType.DMA,
    ],
)
def _deg_kernel(ei_hbm, out_hbm, idx_v, ones_v, zb_v, deg_sh,
                sem0, sem1, sem2, sem3):
    cid = lax.axis_index("c")
    sid = lax.axis_index("s")
    for i in range(KI // 16):
        ones_v[pl.ds(i * 16, 16)] = jnp.ones((16,), _F32)
    for i in range(RPT // 16):
        zb_v[pl.ds(i * 16, 16)] = jnp.zeros((16,), _F32)
    pltpu.sync_copy(zb_v, deg_sh.at[pl.ds(sid * RPT, RPT)])
    plsc.subcore_barrier()

    @pl.when(cid == 0)
    def _():
        pltpu.sync_copy(ei_hbm.at[1, sid], idx_v)

    @pl.when(cid == 1)
    def _():
        pltpu.sync_copy(ei_hbm.at[0, sid], idx_v)

    sems = (sem0, sem1, sem2, sem3)

    @pl.loop(0, NCH - (NCH % 8), step=8)
    def _(j):
        handles = [
            pltpu.async_copy(
                ones_v, deg_sh.at[idx_v.at[j + s]], sems[s % 4], add=True)
            for s in range(8)
        ]
        for h in handles:
            h.wait()

    for s in range(NCH % 8):
        j = NCH - (NCH % 8) + s
        pltpu.sync_copy(ones_v, deg_sh.at[idx_v.at[j]], add=True)

    plsc.subcore_barrier()
    pltpu.sync_copy(deg_sh.at[pl.ds(sid * RPT, RPT)], zb_v)

    @pl.when(cid == 0)
    def _():
        pltpu.sync_copy(zb_v, out_hbm.at[0, pl.ds(sid * RPT, RPT)])

    @pl.when(cid == 1)
    def _():
        pltpu.sync_copy(zb_v, out_hbm.at[1, pl.ds(sid * RPT, RPT)])


@functools.partial(
    pl.kernel,
    out_type=tuple(
        jax.ShapeDtypeStruct((NPAD, DH), _F32) for _ in range(4)
    ),
    mesh=_MESH,
    scratch_types=[
        pltpu.VMEM((NCH, KI), jnp.int32),
        pltpu.VMEM((NCH, KI), jnp.int32),
        pltpu.VMEM((KI, DH), _F32),
        pltpu.VMEM((KI, DH), _F32),
        pltpu.VMEM((KI, DH), _F32),
        pltpu.VMEM((KI, DH), _F32),
        pltpu.VMEM((WB, DH), _F32),
        pltpu.VMEM_SHARED((NPAD, DH), _F32),
        pltpu.SemaphoreType.DMA,
        pltpu.SemaphoreType.DMA,
        pltpu.SemaphoreType.DMA,
        pltpu.SemaphoreType.DMA,
    ],
    compiler_params=pltpu.CompilerParams(use_tc_tiling_on_sc=False),
)
def _edge_kernel(yf0_hbm, yf1_hbm, yb0_hbm, yb1_hbm, ei_hbm,
                 af0_hbm, af1_hbm, ab0_hbm, ab1_hbm,
                 gidx_v, sidx_v, rows0_v, rows1_v, rows2_v, rows3_v,
                 zb_v, acc_sh, sem0, sem1, sem2, sem3):
    cid = lax.axis_index("c")
    sid = lax.axis_index("s")
    for i in range(WB):
        for j in range(DH // 16):
            zb_v[i, pl.ds(j * 16, 16)] = jnp.zeros((16,), _F32)

    def one_task(grow, srow, tables, outs):
        hg = pltpu.async_copy(ei_hbm.at[grow, sid], gidx_v, sem0)
        hs = pltpu.async_copy(ei_hbm.at[srow, sid], sidx_v, sem1)
        for t in range(RPT // WB):
            pltpu.sync_copy(zb_v, acc_sh.at[pl.ds(sid * RPT + t * WB, WB)])
        plsc.subcore_barrier()
        hg.wait()
        hs.wait()

        bufs = (rows0_v, rows1_v, rows2_v, rows3_v)
        sems = (sem0, sem1, sem2, sem3)

        def run(table_hbm):
            for s in range(4):
                pltpu.async_copy(table_hbm.at[gidx_v.at[s]], bufs[s], sems[s])

            @pl.loop(0, NCH - (NCH % 4), step=4)
            def _(j):
                for s in range(4):
                    pltpu.make_async_copy(
                        table_hbm.at[gidx_v.at[j + s]], bufs[s], sems[s]).wait()
                    pltpu.sync_copy(
                        bufs[s], acc_sh.at[sidx_v.at[j + s]], add=True)

                    @pl.when(j + s + 4 < NCH)
                    def _():
                        pltpu.async_copy(
                            table_hbm.at[gidx_v.at[j + s + 4]], bufs[s], sems[s])

            for s in range(NCH % 4):
                j = NCH - (NCH % 4) + s
                pltpu.make_async_copy(
                    table_hbm.at[gidx_v.at[j]], bufs[s], sems[s]).wait()
                pltpu.sync_copy(bufs[s], acc_sh.at[sidx_v.at[j]], add=True)

        @pl.when(cid == 0)
        def _():
            run(tables[0])

        @pl.when(cid == 1)
        def _():
            run(tables[1])

        plsc.subcore_barrier()

        def writeback(out_hbm):
            for t in range(RPT // WB):
                off = sid * RPT + t * WB
                pltpu.sync_copy(acc_sh.at[pl.ds(off, WB)], rows0_v)
                pltpu.sync_copy(rows0_v, out_hbm.at[pl.ds(off, WB)])

        @pl.when(cid == 0)
        def _():
            writeback(outs[0])

        @pl.when(cid == 1)
        def _():
            writeback(outs[1])

    one_task(0, 1, (yf0_hbm, yf1_hbm), (af0_hbm, af1_hbm))
    one_task(1, 0, (yb0_hbm, yb1_hbm), (ab0_hbm, ab1_hbm))


def _mm_body(x_ref, wf_ref, wb_ref, deg_ref,
             yf0_ref, yf1_ref, yb0_ref, yb1_ref):
    dinv = lax.rsqrt(deg_ref[...] + 1.0)
    x = x_ref[...]
    yf = jnp.dot(x, wf_ref[...], preferred_element_type=_F32) * dinv[:, 0:1]
    yb = jnp.dot(x, wb_ref[...], preferred_element_type=_F32) * dinv[:, 1:2]
    yf0_ref[...] = yf[:, :DH]
    yf1_ref[...] = yf[:, DH:]
    yb0_ref[...] = yb[:, :DH]
    yb1_ref[...] = yb[:, DH:]


BN = 1000
_G = N // BN

_mm_call = pl.pallas_call(
    _mm_body,
    grid=(_G,),
    in_specs=[
        pl.BlockSpec((BN, D), lambda i: (i, 0)),
        pl.BlockSpec((D, D), lambda i: (0, 0)),
        pl.BlockSpec((D, D), lambda i: (0, 0)),
        pl.BlockSpec((BN, 2), lambda i: (i, 0)),
    ],
    out_specs=tuple(pl.BlockSpec((BN, DH), lambda i: (i, 0)) for _ in range(4)),
    out_shape=tuple(jax.ShapeDtypeStruct((NPAD, DH), _F32) for _ in range(4)),
)


def _fin_body(af0_ref, af1_ref, ab0_ref, ab1_ref,
              yf0_ref, yf1_ref, yb0_ref, yb1_ref, deg_ref, b_ref, out_ref):
    dinv = lax.rsqrt(deg_ref[...] + 1.0)
    accf = jnp.concatenate([af0_ref[...], af1_ref[...]], axis=1)
    accb = jnp.concatenate([ab0_ref[...], ab1_ref[...]], axis=1)
    yf = jnp.concatenate([yf0_ref[...], yf1_ref[...]], axis=1)
    yb = jnp.concatenate([yb0_ref[...], yb1_ref[...]], axis=1)
    out = (dinv[:, 0:1] * (accf + yf) + dinv[:, 1:2] * (accb + yb) + b_ref[...])
    out_ref[...] = jnp.maximum(out, 0.0)


_fin_call = pl.pallas_call(
    _fin_body,
    grid=(_G,),
    in_specs=[pl.BlockSpec((BN, DH), lambda i: (i, 0)) for _ in range(8)]
    + [
        pl.BlockSpec((BN, 2), lambda i: (i, 0)),
        pl.BlockSpec((1, D), lambda i: (0, 0)),
    ],
    out_specs=pl.BlockSpec((BN, D), lambda i: (i, 0)),
    out_shape=jax.ShapeDtypeStruct((N, D), _F32),
)


@jax.jit
def kernel(x, edge_index, Wf, bf, Wb, bb):
    ei4 = edge_index.reshape(NC, NS, NCH, KI)
    deg2 = _deg_kernel(ei4)
    deg_t = deg2.T
    yf0, yf1, yb0, yb1 = _mm_call(x, Wf, Wb, deg_t)
    af0, af1, ab0, ab1 = _edge_kernel(yf0, yf1, yb0, yb1, ei4)
    bias = (bf + bb)[None, :]
    return _fin_call(af0, af1, ab0, ab1, yf0, yf1, yb0, yb1, deg_t, bias)


# speedup vs baseline: 1.2669x; 1.0003x over previous
import functools

import jax
import jax.numpy as jnp
from jax import lax
from jax.experimental import pallas as pl
from jax.experimental.pallas import tpu as pltpu
from jax.experimental.pallas import tpu_sc as plsc

N = 10000
E = 320000
D = 128
DH = D // 2

NC = 2
NS = 16
NPAD = 10240
RPT = NPAD // NS
EPT = E // NS
KI = 80
NCH = EPT // KI
WB = 80
_MESH = plsc.VectorSubcoreMesh(
    core_axis_name="c", subcore_axis_name="s", num_cores=NC, num_subcores=NS
)
_F32 = jnp.float32


@functools.partial(
    pl.kernel,
    out_type=jax.ShapeDtypeStruct((NC, NPAD), _F32),
    mesh=_MESH,
    scratch_types=[
        pltpu.VMEM((NCH, KI), jnp.int32),
        pltpu.VMEM((KI,), _F32),
        pltpu.VMEM((RPT,), _F32),
        pltpu.VMEM_SHARED((NPAD,), _F32),
        pltpu.SemaphoreType.DMA,
        pltpu.SemaphoreType.DMA,
        pltpu.SemaphoreType.DMA,
        pltpu.SemaphoreType.DMA,
    ],
)
def _deg_kernel(ei_hbm, out_hbm, idx_v, ones_v, zb_v, deg_sh,
                sem0, sem1, sem2, sem3):
    cid = lax.axis_index("c")
    sid = lax.axis_index("s")
    @pl.when(cid == 0)
    def _():
        pltpu.async_copy(ei_hbm.at[1, sid], idx_v, sem0)

    @pl.when(cid == 1)
    def _():
        pltpu.async_copy(ei_hbm.at[0, sid], idx_v, sem0)

    for i in range(KI // 16):
        ones_v[pl.ds(i * 16, 16)] = jnp.ones((16,), _F32)
    for i in range(RPT // 16):
        zb_v[pl.ds(i * 16, 16)] = jnp.zeros((16,), _F32)
    pltpu.sync_copy(zb_v, deg_sh.at[pl.ds(sid * RPT, RPT)])
    plsc.subcore_barrier()
    pltpu.make_async_copy(ei_hbm.at[0, sid], idx_v, sem0).wait()

    sems = (sem0, sem1, sem2, sem3)

    @pl.loop(0, NCH - (NCH % 8), step=8)
    def _(j):
        handles = [
            pltpu.async_copy(
                ones_v, deg_sh.at[idx_v.at[j + s]], sems[s % 4], add=True)
            for s in range(8)
        ]
        for h in handles:
            h.wait()

    for s in range(NCH % 8):
        j = NCH - (NCH % 8) + s
        pltpu.sync_copy(ones_v, deg_sh.at[idx_v.at[j]], add=True)

    plsc.subcore_barrier()
    pltpu.sync_copy(deg_sh.at[pl.ds(sid * RPT, RPT)], zb_v)

    @pl.when(cid == 0)
    def _():
        pltpu.sync_copy(zb_v, out_hbm.at[0, pl.ds(sid * RPT, RPT)])

    @pl.when(cid == 1)
    def _():
        pltpu.sync_copy(zb_v, out_hbm.at[1, pl.ds(sid * RPT, RPT)])


@functools.partial(
    pl.kernel,
    out_type=tuple(
        jax.ShapeDtypeStruct((NPAD, DH), _F32) for _ in range(4)
    ),
    mesh=_MESH,
    scratch_types=[
        pltpu.VMEM((NCH, KI), jnp.int32),
        pltpu.VMEM((NCH, KI), jnp.int32),
        pltpu.VMEM((KI, DH), _F32),
        pltpu.VMEM((KI, DH), _F32),
        pltpu.VMEM((KI, DH), _F32),
        pltpu.VMEM((KI, DH), _F32),
        pltpu.VMEM((WB, DH), _F32),
        pltpu.VMEM_SHARED((NPAD, DH), _F32),
        pltpu.SemaphoreType.DMA,
        pltpu.SemaphoreType.DMA,
        pltpu.SemaphoreType.DMA,
        pltpu.SemaphoreType.DMA,
    ],
    compiler_params=pltpu.CompilerParams(use_tc_tiling_on_sc=False),
)
def _edge_kernel(yf0_hbm, yf1_hbm, yb0_hbm, yb1_hbm, ei_hbm,
                 af0_hbm, af1_hbm, ab0_hbm, ab1_hbm,
                 gidx_v, sidx_v, rows0_v, rows1_v, rows2_v, rows3_v,
                 zb_v, acc_sh, sem0, sem1, sem2, sem3):
    cid = lax.axis_index("c")
    sid = lax.axis_index("s")
    for i in range(WB):
        for j in range(DH // 16):
            zb_v[i, pl.ds(j * 16, 16)] = jnp.zeros((16,), _F32)

    def one_task(grow, srow, tables, outs):
        hg = pltpu.async_copy(ei_hbm.at[grow, sid], gidx_v, sem0)
        hs = pltpu.async_copy(ei_hbm.at[srow, sid], sidx_v, sem1)
        for t in range(RPT // WB):
            pltpu.sync_copy(zb_v, acc_sh.at[pl.ds(sid * RPT + t * WB, WB)])
        plsc.subcore_barrier()
        hg.wait()
        hs.wait()

        bufs = (rows0_v, rows1_v, rows2_v, rows3_v)
        sems = (sem0, sem1, sem2, sem3)

        def run(table_hbm):
            for s in range(4):
                pltpu.async_copy(table_hbm.at[gidx_v.at[s]], bufs[s], sems[s])

            @pl.loop(0, NCH - (NCH % 4), step=4)
            def _(j):
                for s in range(4):
                    pltpu.make_async_copy(
                        table_hbm.at[gidx_v.at[j + s]], bufs[s], sems[s]).wait()
                    pltpu.sync_copy(
                        bufs[s], acc_sh.at[sidx_v.at[j + s]], add=True)

                    @pl.when(j + s + 4 < NCH)
                    def _():
                        pltpu.async_copy(
                            table_hbm.at[gidx_v.at[j + s + 4]], bufs[s], sems[s])

            for s in range(NCH % 4):
                j = NCH - (NCH % 4) + s
                pltpu.make_async_copy(
                    table_hbm.at[gidx_v.at[j]], bufs[s], sems[s]).wait()
                pltpu.sync_copy(bufs[s], acc_sh.at[sidx_v.at[j]], add=True)

        @pl.when(cid == 0)
        def _():
            run(tables[0])

        @pl.when(cid == 1)
        def _():
            run(tables[1])

        plsc.subcore_barrier()

        def writeback(out_hbm):
            for t in range(RPT // WB):
                off = sid * RPT + t * WB
                pltpu.sync_copy(acc_sh.at[pl.ds(off, WB)], rows0_v)
                pltpu.sync_copy(rows0_v, out_hbm.at[pl.ds(off, WB)])

        @pl.when(cid == 0)
        def _():
            writeback(outs[0])

        @pl.when(cid == 1)
        def _():
            writeback(outs[1])

    one_task(0, 1, (yf0_hbm, yf1_hbm), (af0_hbm, af1_hbm))
    one_task(1, 0, (yb0_hbm, yb1_hbm), (ab0_hbm, ab1_hbm))


def _mm_body(x_ref, wf_ref, wb_ref, deg_ref,
             yf0_ref, yf1_ref, yb0_ref, yb1_ref):
    dinv = lax.rsqrt(deg_ref[...] + 1.0)
    x = x_ref[...]
    yf = jnp.dot(x, wf_ref[...], preferred_element_type=_F32) * dinv[:, 0:1]
    yb = jnp.dot(x, wb_ref[...], preferred_element_type=_F32) * dinv[:, 1:2]
    yf0_ref[...] = yf[:, :DH]
    yf1_ref[...] = yf[:, DH:]
    yb0_ref[...] = yb[:, :DH]
    yb1_ref[...] = yb[:, DH:]


BN = 1000
_G = N // BN

_mm_call = pl.pallas_call(
    _mm_body,
    grid=(_G,),
    in_specs=[
        pl.BlockSpec((BN, D), lambda i: (i, 0)),
        pl.BlockSpec((D, D), lambda i: (0, 0)),
        pl.BlockSpec((D, D), lambda i: (0, 0)),
        pl.BlockSpec((BN, 2), lambda i: (i, 0)),
    ],
    out_specs=tuple(pl.BlockSpec((BN, DH), lambda i: (i, 0)) for _ in range(4)),
    out_shape=tuple(jax.ShapeDtypeStruct((NPAD, DH), _F32) for _ in range(4)),
)


def _fin_body(af0_ref, af1_ref, ab0_ref, ab1_ref,
              yf0_ref, yf1_ref, yb0_ref, yb1_ref, deg_ref, b_ref, out_ref):
    dinv = lax.rsqrt(deg_ref[...] + 1.0)
    accf = jnp.concatenate([af0_ref[...], af1_ref[...]], axis=1)
    accb = jnp.concatenate([ab0_ref[...], ab1_ref[...]], axis=1)
    yf = jnp.concatenate([yf0_ref[...], yf1_ref[...]], axis=1)
    yb = jnp.concatenate([yb0_ref[...], yb1_ref[...]], axis=1)
    out = (dinv[:, 0:1] * (accf + yf) + dinv[:, 1:2] * (accb + yb) + b_ref[...])
    out_ref[...] = jnp.maximum(out, 0.0)


_fin_call = pl.pallas_call(
    _fin_body,
    grid=(_G,),
    in_specs=[pl.BlockSpec((BN, DH), lambda i: (i, 0)) for _ in range(8)]
    + [
        pl.BlockSpec((BN, 2), lambda i: (i, 0)),
        pl.BlockSpec((1, D), lambda i: (0, 0)),
    ],
    out_specs=pl.BlockSpec((BN, D), lambda i: (i, 0)),
    out_shape=jax.ShapeDtypeStruct((N, D), _F32),
)


@jax.jit
def kernel(x, edge_index, Wf, bf, Wb, bb):
    ei4 = edge_index.reshape(NC, NS, NCH, KI)
    deg2 = _deg_kernel(ei4)
    deg_t = deg2.T
    yf0, yf1, yb0, yb1 = _mm_call(x, Wf, Wb, deg_t)
    af0, af1, ab0, ab1 = _edge_kernel(yf0, yf1, yb0, yb1, ei4)
    bias = (bf + bb)[None, :]
    return _fin_call(af0, af1, ab0, ab1, yf0, yf1, yb0, yb1, deg_t, bias)


# speedup vs baseline: 1.2681x; 1.0010x over previous
import functools

import jax
import jax.numpy as jnp
from jax import lax
from jax.experimental import pallas as pl
from jax.experimental.pallas import tpu as pltpu
from jax.experimental.pallas import tpu_sc as plsc

N = 10000
E = 320000
D = 128
DH = D // 2

NC = 2
NS = 16
NPAD = 10240
RPT = NPAD // NS
EPT = E // NS
KI = 80
NCH = EPT // KI
WB = 80
_MESH = plsc.VectorSubcoreMesh(
    core_axis_name="c", subcore_axis_name="s", num_cores=NC, num_subcores=NS
)
_F32 = jnp.float32


@functools.partial(
    pl.kernel,
    out_type=jax.ShapeDtypeStruct((NC, NPAD), _F32),
    mesh=_MESH,
    scratch_types=[
        pltpu.VMEM((NCH, KI), jnp.int32),
        pltpu.VMEM((KI,), _F32),
        pltpu.VMEM((RPT,), _F32),
        pltpu.VMEM_SHARED((NPAD,), _F32),
        pltpu.SemaphoreType.DMA,
        pltpu.SemaphoreType.DMA,
        pltpu.SemaphoreType.DMA,
        pltpu.SemaphoreType.DMA,
    ],
)
def _deg_kernel(ei_hbm, out_hbm, idx_v, ones_v, zb_v, deg_sh,
                sem0, sem1, sem2, sem3):
    cid = lax.axis_index("c")
    sid = lax.axis_index("s")
    @pl.when(cid == 0)
    def _():
        pltpu.async_copy(ei_hbm.at[1, sid], idx_v, sem0)

    @pl.when(cid == 1)
    def _():
        pltpu.async_copy(ei_hbm.at[0, sid], idx_v, sem0)

    for i in range(KI // 16):
        ones_v[pl.ds(i * 16, 16)] = jnp.ones((16,), _F32)
    for i in range(RPT // 16):
        zb_v[pl.ds(i * 16, 16)] = jnp.zeros((16,), _F32)
    pltpu.sync_copy(zb_v, deg_sh.at[pl.ds(sid * RPT, RPT)])
    plsc.subcore_barrier()
    pltpu.make_async_copy(ei_hbm.at[0, sid], idx_v, sem0).wait()

    sems = (sem0, sem1, sem2, sem3)

    @pl.loop(0, NCH - (NCH % 8), step=8)
    def _(j):
        handles = [
            pltpu.async_copy(
                ones_v, deg_sh.at[idx_v.at[j + s]], sems[s % 4], add=True)
            for s in range(8)
        ]
        for h in handles:
            h.wait()

    for s in range(NCH % 8):
        j = NCH - (NCH % 8) + s
        pltpu.sync_copy(ones_v, deg_sh.at[idx_v.at[j]], add=True)

    plsc.subcore_barrier()
    pltpu.sync_copy(deg_sh.at[pl.ds(sid * RPT, RPT)], zb_v)

    @pl.when(cid == 0)
    def _():
        pltpu.sync_copy(zb_v, out_hbm.at[0, pl.ds(sid * RPT, RPT)])

    @pl.when(cid == 1)
    def _():
        pltpu.sync_copy(zb_v, out_hbm.at[1, pl.ds(sid * RPT, RPT)])


@functools.partial(
    pl.kernel,
    out_type=tuple(
        jax.ShapeDtypeStruct((NPAD, DH), _F32) for _ in range(4)
    ),
    mesh=_MESH,
    scratch_types=[
        pltpu.VMEM((NCH, KI), jnp.int32),
        pltpu.VMEM((NCH, KI), jnp.int32),
        pltpu.VMEM((KI, DH), _F32),
        pltpu.VMEM((KI, DH), _F32),
        pltpu.VMEM((KI, DH), _F32),
        pltpu.VMEM((KI, DH), _F32),
        pltpu.VMEM((WB, DH), _F32),
        pltpu.VMEM_SHARED((NPAD, DH), _F32),
        pltpu.SemaphoreType.DMA,
        pltpu.SemaphoreType.DMA,
        pltpu.SemaphoreType.DMA,
        pltpu.SemaphoreType.DMA,
    ],
    compiler_params=pltpu.CompilerParams(use_tc_tiling_on_sc=False),
)
def _edge_kernel(yf0_hbm, yf1_hbm, yb0_hbm, yb1_hbm, ei_hbm,
                 af0_hbm, af1_hbm, ab0_hbm, ab1_hbm,
                 gidx_v, sidx_v, rows0_v, rows1_v, rows2_v, rows3_v,
                 zb_v, acc_sh, sem0, sem1, sem2, sem3):
    cid = lax.axis_index("c")
    sid = lax.axis_index("s")
    for i in range(WB):
        for j in range(DH // 16):
            zb_v[i, pl.ds(j * 16, 16)] = jnp.zeros((16,), _F32)

    bufs = (rows0_v, rows1_v, rows2_v, rows3_v)
    sems = (sem0, sem1, sem2, sem3)

    def one_task(grow, srow, tables, outs, zero_first, rezero):
        hg = pltpu.async_copy(ei_hbm.at[grow, sid], gidx_v, sem0)
        hs = pltpu.async_copy(ei_hbm.at[srow, sid], sidx_v, sem1)
        if zero_first:
            for t in range(RPT // WB):
                pltpu.sync_copy(zb_v, acc_sh.at[pl.ds(sid * RPT + t * WB, WB)])
        plsc.subcore_barrier()
        hg.wait()
        hs.wait()

        def run(table_hbm):
            for s in range(4):
                pltpu.async_copy(table_hbm.at[gidx_v.at[s]], bufs[s], sems[s])

            @pl.loop(0, NCH - (NCH % 4), step=4)
            def _(j):
                for s in range(4):
                    pltpu.make_async_copy(
                        table_hbm.at[gidx_v.at[j + s]], bufs[s], sems[s]).wait()
                    pltpu.sync_copy(
                        bufs[s], acc_sh.at[sidx_v.at[j + s]], add=True)

                    @pl.when(j + s + 4 < NCH)
                    def _():
                        pltpu.async_copy(
                            table_hbm.at[gidx_v.at[j + s + 4]], bufs[s], sems[s])

            for s in range(NCH % 4):
                j = NCH - (NCH % 4) + s
                pltpu.make_async_copy(
                    table_hbm.at[gidx_v.at[j]], bufs[s], sems[s]).wait()
                pltpu.sync_copy(bufs[s], acc_sh.at[sidx_v.at[j]], add=True)

        @pl.when(cid == 0)
        def _():
            run(tables[0])

        @pl.when(cid == 1)
        def _():
            run(tables[1])

        plsc.subcore_barrier()

        def writeback(out_hbm):
            handles = [None, None]
            for t in range(RPT // WB):
                b = t % 2
                off = sid * RPT + t * WB
                if handles[b] is not None:
                    handles[b].wait()
                pltpu.sync_copy(acc_sh.at[pl.ds(off, WB)], bufs[b])
                if rezero:
                    pltpu.sync_copy(zb_v, acc_sh.at[pl.ds(off, WB)])
                handles[b] = pltpu.async_copy(
                    bufs[b], out_hbm.at[pl.ds(off, WB)], sems[b])
            handles[0].wait()
            handles[1].wait()

        @pl.when(cid == 0)
        def _():
            writeback(outs[0])

        @pl.when(cid == 1)
        def _():
            writeback(outs[1])

    one_task(0, 1, (yf0_hbm, yf1_hbm), (af0_hbm, af1_hbm), True, True)
    one_task(1, 0, (yb0_hbm, yb1_hbm), (ab0_hbm, ab1_hbm), False, False)


def _mm_body(x_ref, wf_ref, wb_ref, deg_ref,
             yf0_ref, yf1_ref, yb0_ref, yb1_ref):
    dinv = lax.rsqrt(deg_ref[...] + 1.0)
    x = x_ref[...]
    yf = jnp.dot(x, wf_ref[...], preferred_element_type=_F32) * dinv[:, 0:1]
    yb = jnp.dot(x, wb_ref[...], preferred_element_type=_F32) * dinv[:, 1:2]
    yf0_ref[...] = yf[:, :DH]
    yf1_ref[...] = yf[:, DH:]
    yb0_ref[...] = yb[:, :DH]
    yb1_ref[...] = yb[:, DH:]


BN = 1000
_G = N // BN

_mm_call = pl.pallas_call(
    _mm_body,
    grid=(_G,),
    in_specs=[
        pl.BlockSpec((BN, D), lambda i: (i, 0)),
        pl.BlockSpec((D, D), lambda i: (0, 0)),
        pl.BlockSpec((D, D), lambda i: (0, 0)),
        pl.BlockSpec((BN, 2), lambda i: (i, 0)),
    ],
    out_specs=tuple(pl.BlockSpec((BN, DH), lambda i: (i, 0)) for _ in range(4)),
    out_shape=tuple(jax.ShapeDtypeStruct((NPAD, DH), _F32) for _ in range(4)),
)


def _fin_body(af0_ref, af1_ref, ab0_ref, ab1_ref,
              yf0_ref, yf1_ref, yb0_ref, yb1_ref, deg_ref, b_ref, out_ref):
    dinv = lax.rsqrt(deg_ref[...] + 1.0)
    accf = jnp.concatenate([af0_ref[...], af1_ref[...]], axis=1)
    accb = jnp.concatenate([ab0_ref[...], ab1_ref[...]], axis=1)
    yf = jnp.concatenate([yf0_ref[...], yf1_ref[...]], axis=1)
    yb = jnp.concatenate([yb0_ref[...], yb1_ref[...]], axis=1)
    out = (dinv[:, 0:1] * (accf + yf) + dinv[:, 1:2] * (accb + yb) + b_ref[...])
    out_ref[...] = jnp.maximum(out, 0.0)


_fin_call = pl.pallas_call(
    _fin_body,
    grid=(_G,),
    in_specs=[pl.BlockSpec((BN, DH), lambda i: (i, 0)) for _ in range(8)]
    + [
        pl.BlockSpec((BN, 2), lambda i: (i, 0)),
        pl.BlockSpec((1, D), lambda i: (0, 0)),
    ],
    out_specs=pl.BlockSpec((BN, D), lambda i: (i, 0)),
    out_shape=jax.ShapeDtypeStruct((N, D), _F32),
)


@jax.jit
def kernel(x, edge_index, Wf, bf, Wb, bb):
    ei4 = edge_index.reshape(NC, NS, NCH, KI)
    deg2 = _deg_kernel(ei4)
    deg_t = deg2.T
    yf0, yf1, yb0, yb1 = _mm_call(x, Wf, Wb, deg_t)
    af0, af1, ab0, ab1 = _edge_kernel(yf0, yf1, yb0, yb1, ei4)
    bias = (bf + bb)[None, :]
    return _fin_call(af0, af1, ab0, ab1, yf0, yf1, yb0, yb1, deg_t, bias)


# speedup vs baseline: 1.2838x; 1.0124x over previous
import functools

import jax
import jax.numpy as jnp
from jax import lax
from jax.experimental import pallas as pl
from jax.experimental.pallas import tpu as pltpu
from jax.experimental.pallas import tpu_sc as plsc

N = 10000
E = 320000
D = 128
DH = D // 2

NC = 2
NS = 16
NPAD = 10240
RPT = NPAD // NS
EPT = E // NS
KI = 80
NCH = EPT // KI
WB = 80
_MESH = plsc.VectorSubcoreMesh(
    core_axis_name="c", subcore_axis_name="s", num_cores=NC, num_subcores=NS
)
_F32 = jnp.float32


@functools.partial(
    pl.kernel,
    out_type=jax.ShapeDtypeStruct((NC, NPAD), _F32),
    mesh=_MESH,
    scratch_types=[
        pltpu.VMEM((NCH, KI), jnp.int32),
        pltpu.VMEM((KI,), _F32),
        pltpu.VMEM((RPT,), _F32),
        pltpu.VMEM_SHARED((NPAD,), _F32),
        pltpu.SemaphoreType.DMA,
        pltpu.SemaphoreType.DMA,
        pltpu.SemaphoreType.DMA,
        pltpu.SemaphoreType.DMA,
    ],
)
def _deg_kernel(ei_hbm, out_hbm, idx_v, ones_v, zb_v, deg_sh,
                sem0, sem1, sem2, sem3):
    cid = lax.axis_index("c")
    sid = lax.axis_index("s")
    @pl.when(cid == 0)
    def _():
        pltpu.async_copy(ei_hbm.at[1, sid], idx_v, sem0)

    @pl.when(cid == 1)
    def _():
        pltpu.async_copy(ei_hbm.at[0, sid], idx_v, sem0)

    for i in range(KI // 16):
        ones_v[pl.ds(i * 16, 16)] = jnp.ones((16,), _F32)
    for i in range(RPT // 16):
        zb_v[pl.ds(i * 16, 16)] = jnp.zeros((16,), _F32)
    pltpu.sync_copy(zb_v, deg_sh.at[pl.ds(sid * RPT, RPT)])
    plsc.subcore_barrier()
    pltpu.make_async_copy(ei_hbm.at[0, sid], idx_v, sem0).wait()

    sems = (sem0, sem1, sem2, sem3)

    @pl.loop(0, NCH - (NCH % 8), step=8)
    def _(j):
        handles = [
            pltpu.async_copy(
                ones_v, deg_sh.at[idx_v.at[j + s]], sems[s % 4], add=True)
            for s in range(8)
        ]
        for h in handles:
            h.wait()

    for s in range(NCH % 8):
        j = NCH - (NCH % 8) + s
        pltpu.sync_copy(ones_v, deg_sh.at[idx_v.at[j]], add=True)

    plsc.subcore_barrier()
    pltpu.sync_copy(deg_sh.at[pl.ds(sid * RPT, RPT)], zb_v)

    @pl.when(cid == 0)
    def _():
        pltpu.sync_copy(zb_v, out_hbm.at[0, pl.ds(sid * RPT, RPT)])

    @pl.when(cid == 1)
    def _():
        pltpu.sync_copy(zb_v, out_hbm.at[1, pl.ds(sid * RPT, RPT)])


@functools.partial(
    pl.kernel,
    out_type=tuple(
        jax.ShapeDtypeStruct((NPAD, DH), _F32) for _ in range(4)
    ),
    mesh=_MESH,
    scratch_types=[
        pltpu.VMEM((NCH, KI), jnp.int32),
        pltpu.VMEM((NCH, KI), jnp.int32),
        pltpu.VMEM((KI, DH), _F32),
        pltpu.VMEM((KI, DH), _F32),
        pltpu.VMEM((KI, DH), _F32),
        pltpu.VMEM((KI, DH), _F32),
        pltpu.VMEM((WB, DH), _F32),
        pltpu.VMEM_SHARED((NPAD, DH), _F32),
        pltpu.SemaphoreType.DMA,
        pltpu.SemaphoreType.DMA,
        pltpu.SemaphoreType.DMA,
        pltpu.SemaphoreType.DMA,
    ],
    compiler_params=pltpu.CompilerParams(use_tc_tiling_on_sc=False),
)
def _edge_kernel(yf0_hbm, yf1_hbm, yb0_hbm, yb1_hbm, ei_hbm,
                 af0_hbm, af1_hbm, ab0_hbm, ab1_hbm,
                 gidx_v, sidx_v, rows0_v, rows1_v, rows2_v, rows3_v,
                 zb_v, acc_sh, sem0, sem1, sem2, sem3):
    cid = lax.axis_index("c")
    sid = lax.axis_index("s")
    for i in range(WB):
        for j in range(DH // 16):
            zb_v[i, pl.ds(j * 16, 16)] = jnp.zeros((16,), _F32)

    bufs = (rows0_v, rows1_v, rows2_v, rows3_v)
    sems = (sem0, sem1, sem2, sem3)

    def one_task(grow, srow, tables, outs, zero_first, rezero):
        hg = pltpu.async_copy(ei_hbm.at[grow, sid], gidx_v, sem0)
        hs = pltpu.async_copy(ei_hbm.at[srow, sid], sidx_v, sem1)
        if zero_first:
            for t in range(RPT // WB):
                pltpu.sync_copy(zb_v, acc_sh.at[pl.ds(sid * RPT + t * WB, WB)])
        plsc.subcore_barrier()
        hg.wait()
        hs.wait()

        def run(table_hbm):
            for s in range(4):
                pltpu.async_copy(table_hbm.at[gidx_v.at[s]], bufs[s], sems[s])

            @pl.loop(0, NCH - (NCH % 4), step=4)
            def _(j):
                for s in range(4):
                    pltpu.make_async_copy(
                        table_hbm.at[gidx_v.at[j + s]], bufs[s], sems[s]).wait()
                    pltpu.sync_copy(
                        bufs[s], acc_sh.at[sidx_v.at[j + s]], add=True)

                    @pl.when(j + s + 4 < NCH)
                    def _():
                        pltpu.async_copy(
                            table_hbm.at[gidx_v.at[j + s + 4]], bufs[s], sems[s])

            for s in range(NCH % 4):
                j = NCH - (NCH % 4) + s
                pltpu.make_async_copy(
                    table_hbm.at[gidx_v.at[j]], bufs[s], sems[s]).wait()
                pltpu.sync_copy(bufs[s], acc_sh.at[sidx_v.at[j]], add=True)

        @pl.when(cid == 0)
        def _():
            run(tables[0])

        @pl.when(cid == 1)
        def _():
            run(tables[1])

        plsc.subcore_barrier()

        def writeback(out_hbm):
            handles = [None, None]
            for t in range(RPT // WB):
                b = t % 2
                off = sid * RPT + t * WB
                if handles[b] is not None:
                    handles[b].wait()
                pltpu.sync_copy(acc_sh.at[pl.ds(off, WB)], bufs[b])
                if rezero:
                    pltpu.sync_copy(zb_v, acc_sh.at[pl.ds(off, WB)])
                handles[b] = pltpu.async_copy(
                    bufs[b], out_hbm.at[pl.ds(off, WB)], sems[b])
            handles[0].wait()
            handles[1].wait()

        @pl.when(cid == 0)
        def _():
            writeback(outs[0])

        @pl.when(cid == 1)
        def _():
            writeback(outs[1])

    one_task(0, 1, (yf0_hbm, yf1_hbm), (af0_hbm, af1_hbm), True, True)
    one_task(1, 0, (yb0_hbm, yb1_hbm), (ab0_hbm, ab1_hbm), False, False)


def _mm_body(x_ref, wf_ref, wb_ref, deg_ref,
             yf0_ref, yf1_ref, yb0_ref, yb1_ref):
    dinv = lax.rsqrt(deg_ref[...] + 1.0)
    x = x_ref[...]
    yf = jnp.dot(x, wf_ref[...], preferred_element_type=_F32) * dinv[:, 0:1]
    yb = jnp.dot(x, wb_ref[...], preferred_element_type=_F32) * dinv[:, 1:2]
    yf0_ref[...] = yf[:, :DH]
    yf1_ref[...] = yf[:, DH:]
    yb0_ref[...] = yb[:, :DH]
    yb1_ref[...] = yb[:, DH:]


BN = 2000
_G = N // BN

_mm_call = pl.pallas_call(
    _mm_body,
    grid=(_G,),
    in_specs=[
        pl.BlockSpec((BN, D), lambda i: (i, 0)),
        pl.BlockSpec((D, D), lambda i: (0, 0)),
        pl.BlockSpec((D, D), lambda i: (0, 0)),
        pl.BlockSpec((BN, 2), lambda i: (i, 0)),
    ],
    out_specs=tuple(pl.BlockSpec((BN, DH), lambda i: (i, 0)) for _ in range(4)),
    out_shape=tuple(jax.ShapeDtypeStruct((NPAD, DH), _F32) for _ in range(4)),
)


def _fin_body(af0_ref, af1_ref, ab0_ref, ab1_ref,
              yf0_ref, yf1_ref, yb0_ref, yb1_ref, deg_ref, b_ref, out_ref):
    dinv = lax.rsqrt(deg_ref[...] + 1.0)
    accf = jnp.concatenate([af0_ref[...], af1_ref[...]], axis=1)
    accb = jnp.concatenate([ab0_ref[...], ab1_ref[...]], axis=1)
    yf = jnp.concatenate([yf0_ref[...], yf1_ref[...]], axis=1)
    yb = jnp.concatenate([yb0_ref[...], yb1_ref[...]], axis=1)
    out = (dinv[:, 0:1] * (accf + yf) + dinv[:, 1:2] * (accb + yb) + b_ref[...])
    out_ref[...] = jnp.maximum(out, 0.0)


_fin_call = pl.pallas_call(
    _fin_body,
    grid=(_G,),
    in_specs=[pl.BlockSpec((BN, DH), lambda i: (i, 0)) for _ in range(8)]
    + [
        pl.BlockSpec((BN, 2), lambda i: (i, 0)),
        pl.BlockSpec((1, D), lambda i: (0, 0)),
    ],
    out_specs=pl.BlockSpec((BN, D), lambda i: (i, 0)),
    out_shape=jax.ShapeDtypeStruct((N, D), _F32),
)


@jax.jit
def kernel(x, edge_index, Wf, bf, Wb, bb):
    ei4 = edge_index.reshape(NC, NS, NCH, KI)
    deg2 = _deg_kernel(ei4)
    deg_t = deg2.T
    yf0, yf1, yb0, yb1 = _mm_call(x, Wf, Wb, deg_t)
    af0, af1, ab0, ab1 = _edge_kernel(yf0, yf1, yb0, yb1, ei4)
    bias = (bf + bb)[None, :]
    return _fin_call(af0, af1, ab0, ab1, yf0, yf1, yb0, yb1, deg_t, bias)
